# baseline (speedup 1.0000x reference)
import jax
import jax.numpy as jnp
from jax.experimental import pallas as pl
from jax.experimental.pallas import tpu as pltpu


def _nn_kernel(s1_ref, s2t_ref, out_ref):
    x = s1_ref[0]
    yt = s2t_ref[0]
    x0 = x[:, 0:1]
    x1 = x[:, 1:2]
    x2 = x[:, 2:3]
    y0 = yt[0:1, :]
    y1 = yt[1:2, :]
    y2 = yt[2:3, :]
    inner = (x0 * y0 + x1 * y1) + x2 * y2
    x_sq = (x0 * x0 + x1 * x1) + x2 * x2
    y_sq = (y0 * y0 + y1 * y1) + y2 * y2
    dist = (x_sq + y_sq) - 2.0 * inner
    idx = jnp.argmin(dist, axis=1).astype(jnp.int32)
    out_ref[0, 0, :] = idx


def kernel(S1, S2):
    B, N, D = S1.shape
    M = S2.shape[1]
    TN = 256
    nb = N // TN
    S2t = jnp.transpose(S2, (0, 2, 1))

    out = pl.pallas_call(
        _nn_kernel,
        grid=(B, nb),
        in_specs=[
            pl.BlockSpec((1, TN, D), lambda b, i: (b, i, 0)),
            pl.BlockSpec((1, D, M), lambda b, i: (b, 0, 0)),
        ],
        out_specs=pl.BlockSpec((1, 1, TN), lambda b, i: (b * nb + i, 0, 0)),
        out_shape=jax.ShapeDtypeStruct((B * nb, 1, TN), jnp.int32),
        compiler_params=pltpu.CompilerParams(
            dimension_semantics=("parallel", "parallel"),
        ),
    )(S1, S2t)
    return out.reshape(B, N).astype(jnp.int64)


# speedup vs baseline: 1.4406x; 1.4406x over previous
import jax
import jax.numpy as jnp
from jax.experimental import pallas as pl
from jax.experimental.pallas import tpu as pltpu


def _nn_kernel(s1_ref, s2t_ref, x0_ref, x1_ref, x2_ref,
               y0_ref, y1_ref, y2_ref, out_ref):
    x = s1_ref[0]
    yt = s2t_ref[0]
    x0 = x0_ref[0]
    x1 = x1_ref[0]
    x2 = x2_ref[0]
    y0 = y0_ref[0]
    y1 = y1_ref[0]
    y2 = y2_ref[0]
    inner = jnp.dot(x, yt, preferred_element_type=jnp.float32)
    x_sq = (x0 * x0 + x1 * x1) + x2 * x2
    y_sq = (y0 * y0 + y1 * y1) + y2 * y2
    dist = (x_sq + y_sq) - 2.0 * inner
    idx = jnp.argmin(dist, axis=1).astype(jnp.int32)
    out_ref[0, 0, :] = idx


def kernel(S1, S2):
    B, N, D = S1.shape
    M = S2.shape[1]
    TN = 256
    nb = N // TN
    S2t = jnp.transpose(S2, (0, 2, 1))

    xs = [S1[:, :, d].reshape(B, N, 1) for d in range(3)]
    ys = [S2[:, :, d].reshape(B, 1, M) for d in range(3)]

    x_spec = pl.BlockSpec((1, TN, 1), lambda b, i: (b, i, 0))
    y_spec = pl.BlockSpec((1, 1, M), lambda b, i: (b, 0, 0))

    out = pl.pallas_call(
        _nn_kernel,
        grid=(B, nb),
        in_specs=[
            pl.BlockSpec((1, TN, D), lambda b, i: (b, i, 0)),
            pl.BlockSpec((1, D, M), lambda b, i: (b, 0, 0)),
            x_spec, x_spec, x_spec, y_spec, y_spec, y_spec,
        ],
        out_specs=pl.BlockSpec((1, 1, TN), lambda b, i: (b * nb + i, 0, 0)),
        out_shape=jax.ShapeDtypeStruct((B * nb, 1, TN), jnp.int32),
        compiler_params=pltpu.CompilerParams(
            dimension_semantics=("parallel", "parallel"),
        ),
    )(S1, S2t, *xs, *ys)
    return out.reshape(B, N).astype(jnp.int64)


# speedup vs baseline: 1.4591x; 1.0128x over previous
import jax
import jax.numpy as jnp
from jax.experimental import pallas as pl
from jax.experimental.pallas import tpu as pltpu


def _nn_kernel(s1_ref, s2t_ref, out_ref):
    x = s1_ref[0]
    yt = s2t_ref[0]
    M = yt.shape[1]
    inner = jnp.dot(x, yt, preferred_element_type=jnp.float32)
    x_sq = jnp.sum(x * x, axis=1, keepdims=True)
    y_sq = jnp.sum(yt * yt, axis=0, keepdims=True)
    dist = (x_sq + y_sq) - 2.0 * inner
    m = jnp.min(dist, axis=1, keepdims=True)
    iota = jax.lax.broadcasted_iota(jnp.int32, dist.shape, 1)
    cand = jnp.where(dist <= m, iota, M)
    idx = jnp.min(cand, axis=1).astype(jnp.int32)
    out_ref[0, 0, :] = idx


def kernel(S1, S2):
    B, N, D = S1.shape
    M = S2.shape[1]
    TN = 256
    nb = N // TN
    S2t = jnp.transpose(S2, (0, 2, 1))

    out = pl.pallas_call(
        _nn_kernel,
        grid=(B, nb),
        in_specs=[
            pl.BlockSpec((1, TN, D), lambda b, i: (b, i, 0)),
            pl.BlockSpec((1, D, M), lambda b, i: (b, 0, 0)),
        ],
        out_specs=pl.BlockSpec((1, 1, TN), lambda b, i: (b * nb + i, 0, 0)),
        out_shape=jax.ShapeDtypeStruct((B * nb, 1, TN), jnp.int32),
        compiler_params=pltpu.CompilerParams(
            dimension_semantics=("parallel", "parallel"),
        ),
    )(S1, S2t)
    return out.reshape(B, N).astype(jnp.int64)
